# Initial kernel scaffold; baseline (speedup 1.0000x reference)
import jax
import jax.numpy as jnp
from jax.experimental import pallas as pl


def kernel(x, W):
    raise NotImplementedError("write your pallas kernel here")


# speedup vs baseline: 4.2844x; 4.2844x over previous
import functools

import jax
import jax.numpy as jnp
from jax import lax
from jax.experimental import pallas as pl
from jax.experimental.pallas import tpu as pltpu
from jax.experimental.pallas import tpu_sc as plsc

DIM = 64
NUM_CORES = 2
NUM_SUBCORES = 16
NUM_WORKERS = NUM_CORES * NUM_SUBCORES
CHUNK = 1024


def _embed_sc(idx, W, n_rows):
    per_w = n_rows // NUM_WORKERS
    n_chunks = per_w // CHUNK
    mesh = plsc.VectorSubcoreMesh(core_axis_name="c", subcore_axis_name="s")

    @functools.partial(
        pl.kernel,
        mesh=mesh,
        compiler_params=pltpu.CompilerParams(use_tc_tiling_on_sc=False),
        out_type=jax.ShapeDtypeStruct((n_rows, DIM), jnp.float32),
        scratch_types=[
            pltpu.VMEM((CHUNK,), jnp.int32),
            pltpu.VMEM((CHUNK, DIM), jnp.float32),
            pltpu.SemaphoreType.DMA,
        ],
    )
    def k(W_hbm, idx_hbm, out_hbm, idx_v, rows_v, sem):
        wid = lax.axis_index("s") * NUM_CORES + lax.axis_index("c")
        base = wid * per_w

        def body(i, _):
            off = base + i * CHUNK
            pltpu.sync_copy(idx_hbm.at[pl.ds(off, CHUNK)], idx_v)
            pltpu.async_copy(W_hbm.at[idx_v], rows_v, sem).wait()
            pltpu.sync_copy(rows_v, out_hbm.at[pl.ds(off, CHUNK)])
            return 0

        lax.fori_loop(0, n_chunks, body, 0)

    return k(W, idx)


def kernel(x, W):
    b, s = x.shape
    n_rows = b * s
    idx = x.reshape(n_rows).astype(jnp.int32)
    out = _embed_sc(idx, W.astype(jnp.float32), n_rows)
    return out.reshape(b, s, DIM)


# speedup vs baseline: 4.3431x; 1.0137x over previous
import functools

import jax
import jax.numpy as jnp
from jax import lax
from jax.experimental import pallas as pl
from jax.experimental.pallas import tpu as pltpu
from jax.experimental.pallas import tpu_sc as plsc

DIM = 64
NUM_CORES = 2
NUM_SUBCORES = 16
NUM_WORKERS = NUM_CORES * NUM_SUBCORES
CHUNK = 512
NBUF = 2


def _embed_sc(idx, W, n_rows):
    per_w = n_rows // NUM_WORKERS
    n_chunks = per_w // CHUNK
    assert n_chunks % NBUF == 0 and n_chunks >= 2 * NBUF
    mesh = plsc.VectorSubcoreMesh(core_axis_name="c", subcore_axis_name="s")

    @functools.partial(
        pl.kernel,
        mesh=mesh,
        compiler_params=pltpu.CompilerParams(use_tc_tiling_on_sc=False),
        out_type=jax.ShapeDtypeStruct((n_rows, DIM), jnp.float32),
        scratch_types=[
            pltpu.VMEM((per_w,), jnp.int32),
            [pltpu.VMEM((CHUNK, DIM), jnp.float32)] * NBUF,
            [pltpu.SemaphoreType.DMA] * NBUF,
            [pltpu.SemaphoreType.DMA] * NBUF,
        ],
    )
    def k(W_hbm, idx_hbm, out_hbm, idx_v, rows, gsem, ssem):
        wid = lax.axis_index("s") * NUM_CORES + lax.axis_index("c")
        base = wid * per_w
        pltpu.sync_copy(idx_hbm.at[pl.ds(base, per_w)], idx_v)

        def gather_copy(j, b):
            return pltpu.make_async_copy(
                W_hbm.at[idx_v.at[pl.ds(j * CHUNK, CHUNK)]], rows[b], gsem[b])

        def store_copy(j, b):
            return pltpu.make_async_copy(
                rows[b], out_hbm.at[pl.ds(base + j * CHUNK, CHUNK)], ssem[b])

        for b in range(NBUF):
            gather_copy(b, b).start()

        @pl.loop(0, n_chunks - NBUF, step=NBUF)
        def _(i):
            for b in range(NBUF):
                gather_copy(i + b, b).wait()
                store_copy(i + b, b).start()
            for b in range(NBUF):
                store_copy(i + b, b).wait()
                gather_copy(i + NBUF + b, b).start()

        last = n_chunks - NBUF
        for b in range(NBUF):
            gather_copy(last + b, b).wait()
            store_copy(last + b, b).start()
        for b in range(NBUF):
            store_copy(last + b, b).wait()

    return k(W, idx)


def kernel(x, W):
    b, s = x.shape
    n_rows = b * s
    idx = x.reshape(n_rows).astype(jnp.int32)
    out = _embed_sc(idx, W.astype(jnp.float32), n_rows)
    return out.reshape(b, s, DIM)


# speedup vs baseline: 4.3753x; 1.0074x over previous
import functools

import jax
import jax.numpy as jnp
from jax import lax
from jax.experimental import pallas as pl
from jax.experimental.pallas import tpu as pltpu
from jax.experimental.pallas import tpu_sc as plsc

DIM = 64
NUM_CORES = 2
NUM_SUBCORES = 16
NUM_WORKERS = NUM_CORES * NUM_SUBCORES
NBUF = 4


def _embed_sc(idx, W, b, s):
    seq_per_w = b // NUM_WORKERS
    assert seq_per_w % NBUF == 0 and seq_per_w >= 2 * NBUF
    mesh = plsc.VectorSubcoreMesh(core_axis_name="c", subcore_axis_name="s")

    @functools.partial(
        pl.kernel,
        mesh=mesh,
        compiler_params=pltpu.CompilerParams(use_tc_tiling_on_sc=False),
        out_type=jax.ShapeDtypeStruct((b, s, DIM), jnp.float32),
        scratch_types=[
            pltpu.VMEM((seq_per_w, s), jnp.int32),
            [pltpu.VMEM((s, DIM), jnp.float32)] * NBUF,
            [pltpu.SemaphoreType.DMA] * NBUF,
            [pltpu.SemaphoreType.DMA] * NBUF,
        ],
    )
    def k(W_hbm, idx_hbm, out_hbm, idx_v, rows, gsem, ssem):
        wid = lax.axis_index("s") * NUM_CORES + lax.axis_index("c")
        base = wid * seq_per_w
        pltpu.sync_copy(idx_hbm.at[pl.ds(base, seq_per_w)], idx_v)

        def gather_copy(j, buf):
            return pltpu.make_async_copy(
                W_hbm.at[idx_v.at[j]], rows[buf], gsem[buf])

        def store_copy(j, buf):
            return pltpu.make_async_copy(
                rows[buf], out_hbm.at[base + j], ssem[buf])

        for buf in range(NBUF):
            gather_copy(buf, buf).start()

        @pl.loop(0, seq_per_w - NBUF, step=NBUF)
        def _(i):
            for buf in range(NBUF):
                gather_copy(i + buf, buf).wait()
                store_copy(i + buf, buf).start()
            for buf in range(NBUF):
                store_copy(i + buf, buf).wait()
                gather_copy(i + NBUF + buf, buf).start()

        last = seq_per_w - NBUF
        for buf in range(NBUF):
            gather_copy(last + buf, buf).wait()
            store_copy(last + buf, buf).start()
        for buf in range(NBUF):
            store_copy(last + buf, buf).wait()

    return k(W, idx)


def kernel(x, W):
    b, s = x.shape
    return _embed_sc(x.astype(jnp.int32), W.astype(jnp.float32), b, s)
